# Initial kernel scaffold; baseline (speedup 1.0000x reference)
import jax
import jax.numpy as jnp
from jax.experimental import pallas as pl


def kernel(x, senders, receivers, W_base, b_base, W_edge, b_edge):
    raise NotImplementedError("write your pallas kernel here")


# speedup vs baseline: 3.3453x; 3.3453x over previous
import functools

import jax
import jax.numpy as jnp
from jax import lax
from jax.experimental import pallas as pl
from jax.experimental.pallas import tpu as pltpu
from jax.experimental.pallas import tpu_sc as plsc

N_NODES = 10000
N_EDGES = 320000
D_FEAT = 128
EDGE_DIM = 16

NC = 2
NS = 16
NW = NC * NS
E_PER_W = N_EDGES // NW
CHUNK = 80
N_CHUNKS = E_PER_W // CHUNK

ROWS_BLK = 1000


def _tables_body(x_ref, wb_ref, bb_ref, wc_ref, bc_ref, pr_ref, ps_ref):
    t = jnp.tanh(
        jnp.dot(x_ref[...], wb_ref[...], preferred_element_type=jnp.float32)
        + bb_ref[...]
    )
    p = jnp.dot(t, wc_ref[...], preferred_element_type=jnp.float32) + bc_ref[...]
    pr_ref[...] = p[:, :EDGE_DIM]
    ps_ref[...] = p[:, EDGE_DIM:]


def _node_tables(x, W_base, b_base, W_edge, b_edge):
    w_cat = jnp.concatenate([W_edge[:D_FEAT], W_edge[D_FEAT:]], axis=1)
    b_cat = jnp.concatenate([b_edge, jnp.zeros_like(b_edge)]).reshape(1, 2 * EDGE_DIM)
    grid = (N_NODES // ROWS_BLK,)
    return pl.pallas_call(
        _tables_body,
        grid=grid,
        in_specs=[
            pl.BlockSpec((ROWS_BLK, D_FEAT), lambda i: (i, 0)),
            pl.BlockSpec((D_FEAT, D_FEAT), lambda i: (0, 0)),
            pl.BlockSpec((1, D_FEAT), lambda i: (0, 0)),
            pl.BlockSpec((D_FEAT, 2 * EDGE_DIM), lambda i: (0, 0)),
            pl.BlockSpec((1, 2 * EDGE_DIM), lambda i: (0, 0)),
        ],
        out_specs=[
            pl.BlockSpec((ROWS_BLK, EDGE_DIM), lambda i: (i, 0)),
            pl.BlockSpec((ROWS_BLK, EDGE_DIM), lambda i: (i, 0)),
        ],
        out_shape=[
            jax.ShapeDtypeStruct((N_NODES, EDGE_DIM), jnp.float32),
            jax.ShapeDtypeStruct((N_NODES, EDGE_DIM), jnp.float32),
        ],
    )(x, W_base, b_base.reshape(1, D_FEAT), w_cat, b_cat)


def _edge_body(pr_hbm, ps_hbm, ridx_hbm, sidx_hbm, out_hbm,
               ridx_v, sidx_v, rbuf, sbuf, obuf, sem_r, sem_s):
    wid = lax.axis_index("s") * NC + lax.axis_index("c")
    pltpu.sync_copy(ridx_hbm.at[wid], ridx_v)
    pltpu.sync_copy(sidx_hbm.at[wid], sidx_v)
    out_base = wid * N_CHUNKS

    def chunk_body(j, carry):
        h_r = pltpu.async_copy(pr_hbm.at[ridx_v.at[j]], rbuf, sem_r)
        h_s = pltpu.async_copy(ps_hbm.at[sidx_v.at[j]], sbuf, sem_s)
        h_r.wait()
        h_s.wait()

        def row_body(i, c):
            z = rbuf[i] + sbuf[i]
            e = jnp.exp(z + z)
            obuf[i] = 1.0 - 2.0 / (e + 1.0)
            return c

        lax.fori_loop(0, CHUNK, row_body, 0, unroll=4)
        pltpu.sync_copy(obuf, out_hbm.at[out_base + j])
        return carry

    lax.fori_loop(0, N_CHUNKS, chunk_body, 0)


def _edge_update(pr, ps, ridx3, sidx3):
    mesh = plsc.VectorSubcoreMesh(core_axis_name="c", subcore_axis_name="s")
    f = pl.kernel(
        _edge_body,
        out_type=jax.ShapeDtypeStruct((NW * N_CHUNKS, CHUNK, EDGE_DIM), jnp.float32),
        mesh=mesh,
        scratch_types=[
            pltpu.VMEM((N_CHUNKS, CHUNK), jnp.int32),
            pltpu.VMEM((N_CHUNKS, CHUNK), jnp.int32),
            pltpu.VMEM((CHUNK, EDGE_DIM), jnp.float32),
            pltpu.VMEM((CHUNK, EDGE_DIM), jnp.float32),
            pltpu.VMEM((CHUNK, EDGE_DIM), jnp.float32),
            pltpu.SemaphoreType.DMA,
            pltpu.SemaphoreType.DMA,
        ],
        compiler_params=pltpu.CompilerParams(use_tc_tiling_on_sc=False),
    )
    return f(pr, ps, ridx3, sidx3)


def kernel(x, senders, receivers, W_base, b_base, W_edge, b_edge):
    pr, ps = _node_tables(x, W_base, b_base, W_edge, b_edge)
    ridx3 = receivers.reshape(NW, N_CHUNKS, CHUNK)
    sidx3 = senders.reshape(NW, N_CHUNKS, CHUNK)
    out = _edge_update(pr, ps, ridx3, sidx3)
    return out.reshape(N_EDGES, EDGE_DIM)


# speedup vs baseline: 6.6472x; 1.9870x over previous
import functools

import jax
import jax.numpy as jnp
from jax import lax
from jax.experimental import pallas as pl
from jax.experimental.pallas import tpu as pltpu
from jax.experimental.pallas import tpu_sc as plsc

N_NODES = 10000
N_EDGES = 320000
D_FEAT = 128
EDGE_DIM = 16

NC = 2
NS = 16
NW = NC * NS
E_PER_W = N_EDGES // NW
CHUNK = 80
N_CHUNKS = E_PER_W // CHUNK

ROWS_BLK = 1000


def _tables_body(x_ref, wb_ref, bb_ref, wc_ref, bc_ref, pr_ref, ps_ref):
    t = jnp.tanh(
        jnp.dot(x_ref[...], wb_ref[...], preferred_element_type=jnp.float32)
        + bb_ref[...]
    )
    p = jnp.dot(t, wc_ref[...], preferred_element_type=jnp.float32) + bc_ref[...]
    pr_ref[...] = p[:, :EDGE_DIM]
    ps_ref[...] = p[:, EDGE_DIM:]


def _node_tables(x, W_base, b_base, W_edge, b_edge):
    w_cat = jnp.concatenate([W_edge[:D_FEAT], W_edge[D_FEAT:]], axis=1)
    b_cat = jnp.concatenate([b_edge, jnp.zeros_like(b_edge)]).reshape(1, 2 * EDGE_DIM)
    grid = (N_NODES // ROWS_BLK,)
    return pl.pallas_call(
        _tables_body,
        grid=grid,
        in_specs=[
            pl.BlockSpec((ROWS_BLK, D_FEAT), lambda i: (i, 0)),
            pl.BlockSpec((D_FEAT, D_FEAT), lambda i: (0, 0)),
            pl.BlockSpec((1, D_FEAT), lambda i: (0, 0)),
            pl.BlockSpec((D_FEAT, 2 * EDGE_DIM), lambda i: (0, 0)),
            pl.BlockSpec((1, 2 * EDGE_DIM), lambda i: (0, 0)),
        ],
        out_specs=[
            pl.BlockSpec((ROWS_BLK, EDGE_DIM), lambda i: (i, 0)),
            pl.BlockSpec((ROWS_BLK, EDGE_DIM), lambda i: (i, 0)),
        ],
        out_shape=[
            jax.ShapeDtypeStruct((N_NODES, EDGE_DIM), jnp.float32),
            jax.ShapeDtypeStruct((N_NODES, EDGE_DIM), jnp.float32),
        ],
    )(x, W_base, b_base.reshape(1, D_FEAT), w_cat, b_cat)


NBUF = 2


def _edge_body(pr_hbm, ps_hbm, ridx_hbm, sidx_hbm, out_hbm,
               ridx_v, sidx_v,
               rbuf0, rbuf1, sbuf0, sbuf1, obuf0, obuf1,
               sem_i0, sem_i1, sem_o0, sem_o1):
    rbufs, sbufs, obufs = [rbuf0, rbuf1], [sbuf0, sbuf1], [obuf0, obuf1]
    sem_is, sem_os = [sem_i0, sem_i1], [sem_o0, sem_o1]
    wid = lax.axis_index("s") * NC + lax.axis_index("c")
    pltpu.sync_copy(ridx_hbm.at[wid], ridx_v)
    pltpu.sync_copy(sidx_hbm.at[wid], sidx_v)
    out_base = wid * N_CHUNKS

    def start_in(j, b):
        pltpu.async_copy(pr_hbm.at[ridx_v.at[j]], rbufs[b], sem_is[b])
        pltpu.async_copy(ps_hbm.at[sidx_v.at[j]], sbufs[b], sem_is[b])

    def wait_in(j, b):
        pltpu.make_async_copy(pr_hbm.at[ridx_v.at[j]], rbufs[b], sem_is[b]).wait()
        pltpu.make_async_copy(ps_hbm.at[sidx_v.at[j]], sbufs[b], sem_is[b]).wait()

    def start_out(j, b):
        pltpu.async_copy(obufs[b], out_hbm.at[out_base + j], sem_os[b])

    def wait_out(j, b):
        pltpu.make_async_copy(obufs[b], out_hbm.at[out_base + j], sem_os[b]).wait()

    for b in range(NBUF):
        start_in(b, b)

    @pl.loop(0, N_CHUNKS + (-N_CHUNKS) % NBUF, step=NBUF)
    def outer(j0):
        for b in range(NBUF):
            j = j0 + b

            @pl.when(j < N_CHUNKS)
            def _():
                wait_in(j, b)

                @pl.when(j >= NBUF)
                def _():
                    wait_out(j - NBUF, b)

                rb, sb, ob = rbufs[b], sbufs[b], obufs[b]

                @plsc.parallel_loop(0, CHUNK, unroll=8)
                def rows(i):
                    z = rb[i] + sb[i]
                    e = jnp.exp(z + z)
                    ob[i] = 1.0 - 2.0 / (e + 1.0)

                start_out(j, b)

                @pl.when(j + NBUF < N_CHUNKS)
                def _():
                    start_in(j + NBUF, b)

    for b in range(NBUF):
        last = N_CHUNKS - NBUF + ((b - N_CHUNKS) % NBUF)
        wait_out(last, b)


def _edge_update(pr, ps, ridx3, sidx3):
    mesh = plsc.VectorSubcoreMesh(core_axis_name="c", subcore_axis_name="s")
    f = pl.kernel(
        _edge_body,
        out_type=jax.ShapeDtypeStruct((NW * N_CHUNKS, CHUNK, EDGE_DIM), jnp.float32),
        mesh=mesh,
        scratch_types=[
            pltpu.VMEM((N_CHUNKS, CHUNK), jnp.int32),
            pltpu.VMEM((N_CHUNKS, CHUNK), jnp.int32),
        ] + [pltpu.VMEM((CHUNK, EDGE_DIM), jnp.float32)] * 6 + [
            pltpu.SemaphoreType.DMA,
            pltpu.SemaphoreType.DMA,
            pltpu.SemaphoreType.DMA,
            pltpu.SemaphoreType.DMA,
        ],
        compiler_params=pltpu.CompilerParams(use_tc_tiling_on_sc=False),
    )
    return f(pr, ps, ridx3, sidx3)


def kernel(x, senders, receivers, W_base, b_base, W_edge, b_edge):
    pr, ps = _node_tables(x, W_base, b_base, W_edge, b_edge)
    ridx3 = receivers.reshape(NW, N_CHUNKS, CHUNK)
    sidx3 = senders.reshape(NW, N_CHUNKS, CHUNK)
    out = _edge_update(pr, ps, ridx3, sidx3)
    return out.reshape(N_EDGES, EDGE_DIM)


# speedup vs baseline: 6.6494x; 1.0003x over previous
import functools

import jax
import jax.numpy as jnp
from jax import lax
from jax.experimental import pallas as pl
from jax.experimental.pallas import tpu as pltpu
from jax.experimental.pallas import tpu_sc as plsc

N_NODES = 10000
N_EDGES = 320000
D_FEAT = 128
EDGE_DIM = 16

NC = 2
NS = 16
NW = NC * NS
E_PER_W = N_EDGES // NW
CHUNK = 80
N_CHUNKS = E_PER_W // CHUNK

ROWS_BLK = 1000


def _tables_body(x_ref, wb_ref, bb_ref, wc_ref, bc_ref, pr_ref, ps_ref):
    t = jnp.tanh(
        jnp.dot(x_ref[...], wb_ref[...], preferred_element_type=jnp.float32)
        + bb_ref[...]
    )
    p = jnp.dot(t, wc_ref[...], preferred_element_type=jnp.float32) + bc_ref[...]
    pr_ref[...] = p[:, :EDGE_DIM]
    ps_ref[...] = p[:, EDGE_DIM:]


def _node_tables(x, W_base, b_base, W_edge, b_edge):
    w_cat = jnp.concatenate([W_edge[:D_FEAT], W_edge[D_FEAT:]], axis=1)
    b_cat = jnp.concatenate([b_edge, jnp.zeros_like(b_edge)]).reshape(1, 2 * EDGE_DIM)
    grid = (N_NODES // ROWS_BLK,)
    return pl.pallas_call(
        _tables_body,
        grid=grid,
        in_specs=[
            pl.BlockSpec((ROWS_BLK, D_FEAT), lambda i: (i, 0)),
            pl.BlockSpec((D_FEAT, D_FEAT), lambda i: (0, 0)),
            pl.BlockSpec((1, D_FEAT), lambda i: (0, 0)),
            pl.BlockSpec((D_FEAT, 2 * EDGE_DIM), lambda i: (0, 0)),
            pl.BlockSpec((1, 2 * EDGE_DIM), lambda i: (0, 0)),
        ],
        out_specs=[
            pl.BlockSpec((ROWS_BLK, EDGE_DIM), lambda i: (i, 0)),
            pl.BlockSpec((ROWS_BLK, EDGE_DIM), lambda i: (i, 0)),
        ],
        out_shape=[
            jax.ShapeDtypeStruct((N_NODES, EDGE_DIM), jnp.float32),
            jax.ShapeDtypeStruct((N_NODES, EDGE_DIM), jnp.float32),
        ],
    )(x, W_base, b_base.reshape(1, D_FEAT), w_cat, b_cat)


NBUF = 2


def _edge_body(pr_hbm, ps_hbm, ridx_hbm, sidx_hbm, out_hbm,
               ridx_v, sidx_v,
               rbuf0, rbuf1, sbuf0, sbuf1, obuf0, obuf1,
               sem_i0, sem_i1, sem_o0, sem_o1):
    rbufs, sbufs, obufs = [rbuf0, rbuf1], [sbuf0, sbuf1], [obuf0, obuf1]
    sem_is, sem_os = [sem_i0, sem_i1], [sem_o0, sem_o1]
    wid = lax.axis_index("s") * NC + lax.axis_index("c")
    e_base = wid * E_PER_W
    pltpu.sync_copy(ridx_hbm.at[pl.ds(e_base, E_PER_W)], ridx_v)
    pltpu.sync_copy(sidx_hbm.at[pl.ds(e_base, E_PER_W)], sidx_v)

    def start_in(j, b):
        idx_r = ridx_v.at[pl.ds(j * CHUNK, CHUNK)]
        idx_s = sidx_v.at[pl.ds(j * CHUNK, CHUNK)]
        pltpu.async_copy(pr_hbm.at[idx_r], rbufs[b], sem_is[b])
        pltpu.async_copy(ps_hbm.at[idx_s], sbufs[b], sem_is[b])

    def wait_in(j, b):
        idx_r = ridx_v.at[pl.ds(j * CHUNK, CHUNK)]
        pltpu.make_async_copy(pr_hbm.at[idx_r], rbufs[b], sem_is[b]).wait()
        pltpu.make_async_copy(pr_hbm.at[idx_r], sbufs[b], sem_is[b]).wait()

    def start_out(j, b):
        pltpu.async_copy(obufs[b], out_hbm.at[pl.ds(e_base + j * CHUNK, CHUNK)],
                         sem_os[b])

    def wait_out(j, b):
        pltpu.make_async_copy(obufs[b], out_hbm.at[pl.ds(e_base + j * CHUNK, CHUNK)],
                              sem_os[b]).wait()

    for b in range(NBUF):
        start_in(b, b)

    @pl.loop(0, N_CHUNKS + (-N_CHUNKS) % NBUF, step=NBUF)
    def outer(j0):
        for b in range(NBUF):
            j = j0 + b

            @pl.when(j < N_CHUNKS)
            def _():
                wait_in(j, b)

                @pl.when(j >= NBUF)
                def _():
                    wait_out(j - NBUF, b)

                rb, sb, ob = rbufs[b], sbufs[b], obufs[b]

                @plsc.parallel_loop(0, CHUNK, unroll=8)
                def rows(i):
                    z = rb[i] + sb[i]
                    e = jnp.exp(z + z)
                    ob[i] = 1.0 - 2.0 / (e + 1.0)

                start_out(j, b)

                @pl.when(j + NBUF < N_CHUNKS)
                def _():
                    start_in(j + NBUF, b)

    for b in range(NBUF):
        last = N_CHUNKS - NBUF + ((b - N_CHUNKS) % NBUF)
        wait_out(last, b)


def _edge_update(pr, ps, ridx3, sidx3):
    mesh = plsc.VectorSubcoreMesh(core_axis_name="c", subcore_axis_name="s")
    f = pl.kernel(
        _edge_body,
        out_type=jax.ShapeDtypeStruct((N_EDGES, EDGE_DIM), jnp.float32),
        mesh=mesh,
        scratch_types=[
            pltpu.VMEM((E_PER_W,), jnp.int32),
            pltpu.VMEM((E_PER_W,), jnp.int32),
        ] + [pltpu.VMEM((CHUNK, EDGE_DIM), jnp.float32)] * 6 + [
            pltpu.SemaphoreType.DMA,
            pltpu.SemaphoreType.DMA,
            pltpu.SemaphoreType.DMA,
            pltpu.SemaphoreType.DMA,
        ],
        compiler_params=pltpu.CompilerParams(use_tc_tiling_on_sc=False),
    )
    return f(pr, ps, ridx3, sidx3)


def kernel(x, senders, receivers, W_base, b_base, W_edge, b_edge):
    pr, ps = _node_tables(x, W_base, b_base, W_edge, b_edge)
    return _edge_update(pr, ps, receivers, senders)


# speedup vs baseline: 13.4025x; 2.0156x over previous
import functools

import jax
import jax.numpy as jnp
from jax import lax
from jax.experimental import pallas as pl
from jax.experimental.pallas import tpu as pltpu
from jax.experimental.pallas import tpu_sc as plsc

N_NODES = 10000
N_EDGES = 320000
D_FEAT = 128
EDGE_DIM = 16

NC = 2
NS = 16
NW = NC * NS
CHUNK = 128
N_CHUNKS = N_EDGES // CHUNK
MAX_WCHUNKS = N_CHUNKS // NW + 1
NBUF = 2

ROWS_BLK = 1000


def _tables_body(x_ref, wb_ref, bb_ref, wc_ref, bc_ref, pr_ref, ps_ref):
    t = jnp.tanh(
        jnp.dot(x_ref[...], wb_ref[...], preferred_element_type=jnp.float32)
        + bb_ref[...]
    )
    p = jnp.dot(t, wc_ref[...], preferred_element_type=jnp.float32) + bc_ref[...]
    pr_ref[...] = p[:, :EDGE_DIM]
    ps_ref[...] = p[:, EDGE_DIM:]


def _node_tables(x, W_base, b_base, W_edge, b_edge):
    w_cat = jnp.concatenate([W_edge[:D_FEAT], W_edge[D_FEAT:]], axis=1)
    b_cat = jnp.concatenate([b_edge, jnp.zeros_like(b_edge)]).reshape(1, 2 * EDGE_DIM)
    grid = (N_NODES // ROWS_BLK,)
    return pl.pallas_call(
        _tables_body,
        grid=grid,
        in_specs=[
            pl.BlockSpec((ROWS_BLK, D_FEAT), lambda i: (i, 0)),
            pl.BlockSpec((D_FEAT, D_FEAT), lambda i: (0, 0)),
            pl.BlockSpec((1, D_FEAT), lambda i: (0, 0)),
            pl.BlockSpec((D_FEAT, 2 * EDGE_DIM), lambda i: (0, 0)),
            pl.BlockSpec((1, 2 * EDGE_DIM), lambda i: (0, 0)),
        ],
        out_specs=[
            pl.BlockSpec((ROWS_BLK, EDGE_DIM), lambda i: (i, 0)),
            pl.BlockSpec((ROWS_BLK, EDGE_DIM), lambda i: (i, 0)),
        ],
        out_shape=[
            jax.ShapeDtypeStruct((N_NODES, EDGE_DIM), jnp.float32),
            jax.ShapeDtypeStruct((N_NODES, EDGE_DIM), jnp.float32),
        ],
    )(x, W_base, b_base.reshape(1, D_FEAT), w_cat, b_cat)


def _edge_body(pr_hbm, ps_hbm, ridx_hbm, sidx_hbm, out_hbm,
               ridx_v, sidx_v,
               rbuf0, rbuf1, sbuf0, sbuf1, obuf0, obuf1,
               sem_i0, sem_i1, sem_o0, sem_o1):
    rbufs, sbufs, obufs = [rbuf0, rbuf1], [sbuf0, sbuf1], [obuf0, obuf1]
    sem_is, sem_os = [sem_i0, sem_i1], [sem_o0, sem_o1]
    wid = lax.axis_index("s") * NC + lax.axis_index("c")
    lo_c = lax.shift_right_logical(625 * wid, 3)
    hi_c = lax.shift_right_logical(625 * (wid + 1), 3)
    n_c = hi_c - lo_c
    e_lo = lo_c * CHUNK
    pltpu.sync_copy(ridx_hbm.at[pl.ds(e_lo, MAX_WCHUNKS * CHUNK)], ridx_v)
    pltpu.sync_copy(sidx_hbm.at[pl.ds(e_lo, MAX_WCHUNKS * CHUNK)], sidx_v)
    lanes128 = jnp.arange(EDGE_DIM, dtype=jnp.int32) * CHUNK

    def start_in(k, b):
        idx_r = ridx_v.at[pl.ds(k * CHUNK, CHUNK)]
        idx_s = sidx_v.at[pl.ds(k * CHUNK, CHUNK)]
        pltpu.async_copy(pr_hbm.at[idx_r], rbufs[b], sem_is[b])
        pltpu.async_copy(ps_hbm.at[idx_s], sbufs[b], sem_is[b])

    def wait_in(k, b):
        idx_r = ridx_v.at[pl.ds(k * CHUNK, CHUNK)]
        pltpu.make_async_copy(pr_hbm.at[idx_r], rbufs[b], sem_is[b]).wait()
        pltpu.make_async_copy(pr_hbm.at[idx_r], sbufs[b], sem_is[b]).wait()

    HALF = 8 * CHUNK

    def start_out(k, b):
        c = lo_c + k
        for fr in range(2):
            pltpu.async_copy(obufs[b].at[pl.ds(fr * HALF, HALF)],
                             out_hbm.at[pl.ds((fr * N_CHUNKS + c) * HALF, HALF)],
                             sem_os[b])

    def wait_out(k, b):
        c = lo_c + k
        for fr in range(2):
            pltpu.make_async_copy(obufs[b].at[pl.ds(fr * HALF, HALF)],
                                  out_hbm.at[pl.ds((fr * N_CHUNKS + c) * HALF, HALF)],
                                  sem_os[b]).wait()

    for b in range(NBUF):
        start_in(b, b)

    @pl.loop(0, MAX_WCHUNKS + (-MAX_WCHUNKS) % NBUF, step=NBUF)
    def outer(k0):
        for b in range(NBUF):
            k = k0 + b

            @pl.when(k < n_c)
            def _():
                wait_in(k, b)

                @pl.when(k >= NBUF)
                def _():
                    wait_out(k - NBUF, b)

                rb, sb, ob = rbufs[b], sbufs[b], obufs[b]

                @plsc.parallel_loop(0, CHUNK, unroll=8)
                def rows(i):
                    z = rb[i] + sb[i]
                    e = jnp.exp(z + z)
                    v = 1.0 - 2.0 / (e + 1.0)
                    plsc.store_scatter(ob, [lanes128 + i], v)

                start_out(k, b)

                @pl.when(k + NBUF < n_c)
                def _():
                    start_in(k + NBUF, b)

    for b in range(NBUF):
        k_last = n_c - 1 - ((n_c - 1 - b) % NBUF)
        wait_out(k_last, b)


def _edge_update(pr, ps, ridx, sidx):
    mesh = plsc.VectorSubcoreMesh(core_axis_name="c", subcore_axis_name="s")
    f = pl.kernel(
        _edge_body,
        out_type=jax.ShapeDtypeStruct((2 * N_CHUNKS * 8 * CHUNK,), jnp.float32),
        mesh=mesh,
        scratch_types=[
            pltpu.VMEM((MAX_WCHUNKS * CHUNK,), jnp.int32),
            pltpu.VMEM((MAX_WCHUNKS * CHUNK,), jnp.int32),
        ] + [pltpu.VMEM((CHUNK, EDGE_DIM), jnp.float32)] * 4
          + [pltpu.VMEM((EDGE_DIM * CHUNK,), jnp.float32)] * 2 + [
            pltpu.SemaphoreType.DMA,
            pltpu.SemaphoreType.DMA,
            pltpu.SemaphoreType.DMA,
            pltpu.SemaphoreType.DMA,
        ],
        compiler_params=pltpu.CompilerParams(use_tc_tiling_on_sc=False,
                                             needs_layout_passes=False),
    )
    return f(pr, ps, ridx, sidx)


def kernel(x, senders, receivers, W_base, b_base, W_edge, b_edge):
    pr, ps = _node_tables(x, W_base, b_base, W_edge, b_edge)
    out_flat = _edge_update(pr, ps, receivers, senders)
    out4 = out_flat.reshape(2, N_CHUNKS, 8, CHUNK)
    return out4.transpose(1, 3, 0, 2).reshape(N_EDGES, EDGE_DIM)


# speedup vs baseline: 16.9492x; 1.2646x over previous
import functools

import jax
import jax.numpy as jnp
from jax import lax
from jax.experimental import pallas as pl
from jax.experimental.pallas import tpu as pltpu
from jax.experimental.pallas import tpu_sc as plsc

N_NODES = 10000
N_EDGES = 320000
D_FEAT = 128
EDGE_DIM = 16

NC = 2
NS = 16
NW = NC * NS
CHUNK = 128
N_CHUNKS = N_EDGES // CHUNK
MAX_WCHUNKS = N_CHUNKS // NW + 1
NBUF = 2

ROWS_BLK = 1000


def _tables_body(x_ref, wb_ref, bb_ref, wc_ref, bc_ref, pr_ref, ps_ref):
    t = jnp.tanh(
        jnp.dot(x_ref[...], wb_ref[...], preferred_element_type=jnp.float32)
        + bb_ref[...]
    )
    p = jnp.dot(t, wc_ref[...], preferred_element_type=jnp.float32) + bc_ref[...]
    pr_ref[...] = p[:, :EDGE_DIM]
    ps_ref[...] = p[:, EDGE_DIM:]


def _node_tables(x, W_base, b_base, W_edge, b_edge):
    w_cat = 2.0 * jnp.concatenate([W_edge[:D_FEAT], W_edge[D_FEAT:]], axis=1)
    b_cat = 2.0 * jnp.concatenate(
        [b_edge, jnp.zeros_like(b_edge)]).reshape(1, 2 * EDGE_DIM)
    grid = (N_NODES // ROWS_BLK,)
    return pl.pallas_call(
        _tables_body,
        grid=grid,
        in_specs=[
            pl.BlockSpec((ROWS_BLK, D_FEAT), lambda i: (i, 0)),
            pl.BlockSpec((D_FEAT, D_FEAT), lambda i: (0, 0)),
            pl.BlockSpec((1, D_FEAT), lambda i: (0, 0)),
            pl.BlockSpec((D_FEAT, 2 * EDGE_DIM), lambda i: (0, 0)),
            pl.BlockSpec((1, 2 * EDGE_DIM), lambda i: (0, 0)),
        ],
        out_specs=[
            pl.BlockSpec((ROWS_BLK, EDGE_DIM), lambda i: (i, 0)),
            pl.BlockSpec((ROWS_BLK, EDGE_DIM), lambda i: (i, 0)),
        ],
        out_shape=[
            jax.ShapeDtypeStruct((N_NODES, EDGE_DIM), jnp.float32),
            jax.ShapeDtypeStruct((N_NODES, EDGE_DIM), jnp.float32),
        ],
    )(x, W_base, b_base.reshape(1, D_FEAT), w_cat, b_cat)


def _edge_body(pr_hbm, ps_hbm, ridx_hbm, sidx_hbm, out_hbm,
               ridx_v, sidx_v,
               rbuf0, rbuf1, sbuf0, sbuf1, obuf0, obuf1,
               sem_i0, sem_i1, sem_o0, sem_o1):
    rbufs, sbufs, obufs = [rbuf0, rbuf1], [sbuf0, sbuf1], [obuf0, obuf1]
    sem_is, sem_os = [sem_i0, sem_i1], [sem_o0, sem_o1]
    wid = lax.axis_index("s") * NC + lax.axis_index("c")
    lo_c = lax.shift_right_logical(625 * wid, 3)
    hi_c = lax.shift_right_logical(625 * (wid + 1), 3)
    n_c = hi_c - lo_c
    e_lo = lo_c * CHUNK
    pltpu.sync_copy(ridx_hbm.at[pl.ds(e_lo, MAX_WCHUNKS * CHUNK)], ridx_v)
    pltpu.sync_copy(sidx_hbm.at[pl.ds(e_lo, MAX_WCHUNKS * CHUNK)], sidx_v)
    lanes = jnp.arange(EDGE_DIM, dtype=jnp.int32)

    def start_in(k, b):
        idx_r = ridx_v.at[pl.ds(k * CHUNK, CHUNK)]
        idx_s = sidx_v.at[pl.ds(k * CHUNK, CHUNK)]
        pltpu.async_copy(pr_hbm.at[idx_r], rbufs[b], sem_is[b])
        pltpu.async_copy(ps_hbm.at[idx_s], sbufs[b], sem_is[b])

    def wait_in(k, b):
        idx_r = ridx_v.at[pl.ds(k * CHUNK, CHUNK)]
        pltpu.make_async_copy(pr_hbm.at[idx_r], rbufs[b], sem_is[b]).wait()
        pltpu.make_async_copy(pr_hbm.at[idx_r], sbufs[b], sem_is[b]).wait()

    def start_out(k, b):
        c = lo_c + k
        for fr in range(2):
            pltpu.async_copy(obufs[b].at[pl.ds(fr * 8, 8), pl.ds(0, CHUNK)],
                             out_hbm.at[fr, c], sem_os[b])

    def wait_out(k, b):
        c = lo_c + k
        for fr in range(2):
            pltpu.make_async_copy(obufs[b].at[pl.ds(fr * 8, 8), pl.ds(0, CHUNK)],
                                  out_hbm.at[fr, c], sem_os[b]).wait()

    for b in range(NBUF):
        start_in(b, b)

    @pl.loop(0, MAX_WCHUNKS + (-MAX_WCHUNKS) % NBUF, step=NBUF)
    def outer(k0):
        for b in range(NBUF):
            k = k0 + b

            @pl.when(k < n_c)
            def _():
                wait_in(k, b)

                @pl.when(k >= NBUF)
                def _():
                    wait_out(k - NBUF, b)

                rb, sb, ob = rbufs[b], sbufs[b], obufs[b]

                @plsc.parallel_loop(0, CHUNK, unroll=8)
                def rows(i):
                    e = jnp.exp(rb[i] + sb[i])
                    v = 1.0 - 2.0 / (e + 1.0)
                    plsc.store_scatter(ob, [lanes, jnp.full((EDGE_DIM,), i, jnp.int32)], v)

                start_out(k, b)

                @pl.when(k + NBUF < n_c)
                def _():
                    start_in(k + NBUF, b)

    for b in range(NBUF):
        k_last = n_c - 1 - ((n_c - 1 - b) % NBUF)
        wait_out(k_last, b)


def _edge_update(pr, ps, ridx, sidx):
    mesh = plsc.VectorSubcoreMesh(core_axis_name="c", subcore_axis_name="s")
    f = pl.kernel(
        _edge_body,
        out_type=jax.ShapeDtypeStruct((2, N_CHUNKS, 8, CHUNK), jnp.float32),
        mesh=mesh,
        scratch_types=[
            pltpu.VMEM((MAX_WCHUNKS * CHUNK,), jnp.int32),
            pltpu.VMEM((MAX_WCHUNKS * CHUNK,), jnp.int32),
        ] + [pltpu.VMEM((CHUNK, EDGE_DIM), jnp.float32)] * 4
          + [pltpu.VMEM((EDGE_DIM, CHUNK + 1), jnp.float32)] * 2 + [
            pltpu.SemaphoreType.DMA,
            pltpu.SemaphoreType.DMA,
            pltpu.SemaphoreType.DMA,
            pltpu.SemaphoreType.DMA,
        ],
        compiler_params=pltpu.CompilerParams(use_tc_tiling_on_sc=False,
                                             needs_layout_passes=False),
    )
    return f(pr, ps, ridx, sidx)


def kernel(x, senders, receivers, W_base, b_base, W_edge, b_edge):
    pr, ps = _node_tables(x, W_base, b_base, W_edge, b_edge)
    out4 = _edge_update(pr, ps, receivers, senders)
    return out4.transpose(1, 3, 0, 2).reshape(N_EDGES, EDGE_DIM)


# speedup vs baseline: 19.7344x; 1.1643x over previous
import functools

import jax
import jax.numpy as jnp
from jax import lax
from jax.experimental import pallas as pl
from jax.experimental.pallas import tpu as pltpu
from jax.experimental.pallas import tpu_sc as plsc

N_NODES = 10000
N_EDGES = 320000
D_FEAT = 128
EDGE_DIM = 16

NC = 2
NS = 16
NW = NC * NS
CHUNK = 256
SUB = CHUNK // 128
N_CHUNKS = N_EDGES // CHUNK
N_TCOLS = N_EDGES // 128
MAX_WCHUNKS = N_CHUNKS // NW + 1
NBUF = 2

ROWS_BLK = 1000


def _tables_body(x_ref, wb_ref, bb_ref, wc_ref, bc_ref, pr_ref, ps_ref):
    t = jnp.tanh(
        jnp.dot(x_ref[...], wb_ref[...], preferred_element_type=jnp.float32)
        + bb_ref[...]
    )
    p = jnp.dot(t, wc_ref[...], preferred_element_type=jnp.float32) + bc_ref[...]
    pr_ref[...] = p[:, :EDGE_DIM]
    ps_ref[...] = p[:, EDGE_DIM:]


def _node_tables(x, W_base, b_base, W_edge, b_edge):
    w_cat = 2.0 * jnp.concatenate([W_edge[:D_FEAT], W_edge[D_FEAT:]], axis=1)
    b_cat = 2.0 * jnp.concatenate(
        [b_edge, jnp.zeros_like(b_edge)]).reshape(1, 2 * EDGE_DIM)
    grid = (N_NODES // ROWS_BLK,)
    return pl.pallas_call(
        _tables_body,
        grid=grid,
        in_specs=[
            pl.BlockSpec((ROWS_BLK, D_FEAT), lambda i: (i, 0)),
            pl.BlockSpec((D_FEAT, D_FEAT), lambda i: (0, 0)),
            pl.BlockSpec((1, D_FEAT), lambda i: (0, 0)),
            pl.BlockSpec((D_FEAT, 2 * EDGE_DIM), lambda i: (0, 0)),
            pl.BlockSpec((1, 2 * EDGE_DIM), lambda i: (0, 0)),
        ],
        out_specs=[
            pl.BlockSpec((ROWS_BLK, EDGE_DIM), lambda i: (i, 0)),
            pl.BlockSpec((ROWS_BLK, EDGE_DIM), lambda i: (i, 0)),
        ],
        out_shape=[
            jax.ShapeDtypeStruct((N_NODES, EDGE_DIM), jnp.float32),
            jax.ShapeDtypeStruct((N_NODES, EDGE_DIM), jnp.float32),
        ],
    )(x, W_base, b_base.reshape(1, D_FEAT), w_cat, b_cat)


def _edge_body(pr_hbm, ps_hbm, ridx_hbm, sidx_hbm, out_hbm,
               ridx_v, sidx_v,
               rbuf0, rbuf1, sbuf0, sbuf1, obuf0, obuf1,
               sem_i0, sem_i1, sem_o0, sem_o1):
    rbufs, sbufs, obufs = [rbuf0, rbuf1], [sbuf0, sbuf1], [obuf0, obuf1]
    sem_is, sem_os = [sem_i0, sem_i1], [sem_o0, sem_o1]
    wid = lax.axis_index("s") * NC + lax.axis_index("c")
    lo_c = lax.shift_right_logical(N_CHUNKS * wid, 5)
    hi_c = lax.shift_right_logical(N_CHUNKS * (wid + 1), 5)
    n_c = hi_c - lo_c
    e_lo = lo_c * CHUNK
    pltpu.sync_copy(ridx_hbm.at[pl.ds(e_lo, MAX_WCHUNKS * CHUNK)], ridx_v)
    pltpu.sync_copy(sidx_hbm.at[pl.ds(e_lo, MAX_WCHUNKS * CHUNK)], sidx_v)
    lanes = jnp.arange(EDGE_DIM, dtype=jnp.int32)

    def start_in(k, b):
        idx_r = ridx_v.at[pl.ds(k * CHUNK, CHUNK)]
        idx_s = sidx_v.at[pl.ds(k * CHUNK, CHUNK)]
        pltpu.async_copy(pr_hbm.at[idx_r], rbufs[b], sem_is[b])
        pltpu.async_copy(ps_hbm.at[idx_s], sbufs[b], sem_is[b])

    def wait_in(k, b):
        idx_r = ridx_v.at[pl.ds(k * CHUNK, CHUNK)]
        pltpu.make_async_copy(pr_hbm.at[idx_r], rbufs[b], sem_is[b]).wait()
        pltpu.make_async_copy(pr_hbm.at[idx_r], sbufs[b], sem_is[b]).wait()

    def start_out(k, b):
        c0 = (lo_c + k) * SUB
        for fr in range(2):
            for sub in range(SUB):
                pltpu.async_copy(
                    obufs[b].at[pl.ds(fr * 8, 8), pl.ds(sub * 128, 128)],
                    out_hbm.at[fr, c0 + sub], sem_os[b])

    def wait_out(k, b):
        c0 = (lo_c + k) * SUB
        for fr in range(2):
            for sub in range(SUB):
                pltpu.make_async_copy(
                    obufs[b].at[pl.ds(fr * 8, 8), pl.ds(sub * 128, 128)],
                    out_hbm.at[fr, c0 + sub], sem_os[b]).wait()

    for b in range(NBUF):
        start_in(b, b)

    @pl.loop(0, MAX_WCHUNKS + (-MAX_WCHUNKS) % NBUF, step=NBUF)
    def outer(k0):
        for b in range(NBUF):
            k = k0 + b

            @pl.when(k < n_c)
            def _():
                wait_in(k, b)

                @pl.when(k >= NBUF)
                def _():
                    wait_out(k - NBUF, b)

                rb, sb, ob = rbufs[b], sbufs[b], obufs[b]

                @plsc.parallel_loop(0, CHUNK, unroll=8)
                def rows(i):
                    e = jnp.exp(rb[i] + sb[i])
                    v = 1.0 - 2.0 / (e + 1.0)
                    plsc.store_scatter(
                        ob, [lanes, jnp.full((EDGE_DIM,), i, jnp.int32)], v)

                start_out(k, b)

                @pl.when(k + NBUF < n_c)
                def _():
                    start_in(k + NBUF, b)

    for b in range(NBUF):
        k_last = n_c - 1 - ((n_c - 1 - b) % NBUF)
        wait_out(k_last, b)


def _edge_update(pr, ps, ridx, sidx):
    mesh = plsc.VectorSubcoreMesh(core_axis_name="c", subcore_axis_name="s")
    f = pl.kernel(
        _edge_body,
        out_type=jax.ShapeDtypeStruct((2, N_TCOLS, 8, 128), jnp.float32),
        mesh=mesh,
        scratch_types=[
            pltpu.VMEM((MAX_WCHUNKS * CHUNK,), jnp.int32),
            pltpu.VMEM((MAX_WCHUNKS * CHUNK,), jnp.int32),
        ] + [pltpu.VMEM((CHUNK, EDGE_DIM), jnp.float32)] * 4
          + [pltpu.VMEM((EDGE_DIM, CHUNK + 1), jnp.float32)] * 2 + [
            pltpu.SemaphoreType.DMA,
            pltpu.SemaphoreType.DMA,
            pltpu.SemaphoreType.DMA,
            pltpu.SemaphoreType.DMA,
        ],
        compiler_params=pltpu.CompilerParams(use_tc_tiling_on_sc=False,
                                             needs_layout_passes=False),
    )
    return f(pr, ps, ridx, sidx)


def kernel(x, senders, receivers, W_base, b_base, W_edge, b_edge):
    pr, ps = _node_tables(x, W_base, b_base, W_edge, b_edge)
    out4 = _edge_update(pr, ps, receivers, senders)
    return out4.transpose(1, 3, 0, 2).reshape(N_EDGES, EDGE_DIM)


# speedup vs baseline: 24.3221x; 1.2325x over previous
import functools

import jax
import jax.numpy as jnp
from jax import lax
from jax.experimental import pallas as pl
from jax.experimental.pallas import tpu as pltpu
from jax.experimental.pallas import tpu_sc as plsc

N_NODES = 10000
N_EDGES = 320000
D_FEAT = 128
EDGE_DIM = 16

NC = 2
NS = 16
NW = NC * NS
CHUNK = 512
SUB = CHUNK // 128
N_CHUNKS = N_EDGES // CHUNK
N_TCOLS = N_EDGES // 128
MAX_WCHUNKS = N_CHUNKS // NW + 1
NBUF = 2

ROWS_BLK = 1000


def _tables_body(x_ref, wb_ref, bb_ref, wc_ref, bc_ref, pr_ref, ps_ref):
    t = jnp.tanh(
        jnp.dot(x_ref[...], wb_ref[...], preferred_element_type=jnp.float32)
        + bb_ref[...]
    )
    p = jnp.dot(t, wc_ref[...], preferred_element_type=jnp.float32) + bc_ref[...]
    pr_ref[...] = p[:, :EDGE_DIM]
    ps_ref[...] = p[:, EDGE_DIM:]


def _node_tables(x, W_base, b_base, W_edge, b_edge):
    w_cat = 2.0 * jnp.concatenate([W_edge[:D_FEAT], W_edge[D_FEAT:]], axis=1)
    b_cat = 2.0 * jnp.concatenate(
        [b_edge, jnp.zeros_like(b_edge)]).reshape(1, 2 * EDGE_DIM)
    grid = (N_NODES // ROWS_BLK,)
    return pl.pallas_call(
        _tables_body,
        grid=grid,
        in_specs=[
            pl.BlockSpec((ROWS_BLK, D_FEAT), lambda i: (i, 0)),
            pl.BlockSpec((D_FEAT, D_FEAT), lambda i: (0, 0)),
            pl.BlockSpec((1, D_FEAT), lambda i: (0, 0)),
            pl.BlockSpec((D_FEAT, 2 * EDGE_DIM), lambda i: (0, 0)),
            pl.BlockSpec((1, 2 * EDGE_DIM), lambda i: (0, 0)),
        ],
        out_specs=[
            pl.BlockSpec((ROWS_BLK, EDGE_DIM), lambda i: (i, 0)),
            pl.BlockSpec((ROWS_BLK, EDGE_DIM), lambda i: (i, 0)),
        ],
        out_shape=[
            jax.ShapeDtypeStruct((N_NODES, EDGE_DIM), jnp.float32),
            jax.ShapeDtypeStruct((N_NODES, EDGE_DIM), jnp.float32),
        ],
    )(x, W_base, b_base.reshape(1, D_FEAT), w_cat, b_cat)


def _edge_body(pr_hbm, ps_hbm, ridx_hbm, sidx_hbm, out_hbm,
               ridx_v, sidx_v,
               rbuf0, rbuf1, sbuf0, sbuf1, obuf0, obuf1,
               pr_s, ps_s,
               sem_i0, sem_i1, sem_o0, sem_o1, sem_t):
    rbufs, sbufs, obufs = [rbuf0, rbuf1], [sbuf0, sbuf1], [obuf0, obuf1]
    sem_is, sem_os = [sem_i0, sem_i1], [sem_o0, sem_o1]
    sid = lax.axis_index("s")
    wid = sid * NC + lax.axis_index("c")

    @pl.when(sid == 0)
    def _():
        pltpu.async_copy(pr_hbm, pr_s, sem_t)
        pltpu.async_copy(ps_hbm, ps_s, sem_t)
    lo_c = lax.shift_right_logical(N_CHUNKS * wid, 5)
    hi_c = lax.shift_right_logical(N_CHUNKS * (wid + 1), 5)
    n_c = hi_c - lo_c
    e_lo = lo_c * CHUNK
    pltpu.sync_copy(ridx_hbm.at[pl.ds(e_lo, MAX_WCHUNKS * CHUNK)], ridx_v)
    pltpu.sync_copy(sidx_hbm.at[pl.ds(e_lo, MAX_WCHUNKS * CHUNK)], sidx_v)
    lanes = jnp.arange(EDGE_DIM, dtype=jnp.int32)

    @pl.when(sid == 0)
    def _():
        pltpu.make_async_copy(pr_hbm, pr_s, sem_t).wait()
        pltpu.make_async_copy(ps_hbm, ps_s, sem_t).wait()

    plsc.subcore_barrier()

    def start_in(k, b):
        idx_r = ridx_v.at[pl.ds(k * CHUNK, CHUNK)]
        idx_s = sidx_v.at[pl.ds(k * CHUNK, CHUNK)]
        pltpu.async_copy(pr_s.at[idx_r], rbufs[b], sem_is[b])
        pltpu.async_copy(ps_s.at[idx_s], sbufs[b], sem_is[b])

    def wait_in(k, b):
        idx_r = ridx_v.at[pl.ds(k * CHUNK, CHUNK)]
        pltpu.make_async_copy(pr_s.at[idx_r], rbufs[b], sem_is[b]).wait()
        pltpu.make_async_copy(pr_s.at[idx_r], sbufs[b], sem_is[b]).wait()

    def start_out(k, b):
        c0 = (lo_c + k) * SUB
        for fr in range(2):
            for sub in range(SUB):
                pltpu.async_copy(
                    obufs[b].at[pl.ds(fr * 8, 8), pl.ds(sub * 128, 128)],
                    out_hbm.at[fr, c0 + sub], sem_os[b])

    def wait_out(k, b):
        c0 = (lo_c + k) * SUB
        for fr in range(2):
            for sub in range(SUB):
                pltpu.make_async_copy(
                    obufs[b].at[pl.ds(fr * 8, 8), pl.ds(sub * 128, 128)],
                    out_hbm.at[fr, c0 + sub], sem_os[b]).wait()

    for b in range(NBUF):
        start_in(b, b)

    @pl.loop(0, MAX_WCHUNKS + (-MAX_WCHUNKS) % NBUF, step=NBUF)
    def outer(k0):
        for b in range(NBUF):
            k = k0 + b

            @pl.when(k < n_c)
            def _():
                wait_in(k, b)

                @pl.when(k >= NBUF)
                def _():
                    wait_out(k - NBUF, b)

                rb, sb, ob = rbufs[b], sbufs[b], obufs[b]

                @plsc.parallel_loop(0, CHUNK, unroll=8)
                def rows(i):
                    e = jnp.exp(rb[i] + sb[i])
                    v = 1.0 - 2.0 / (e + 1.0)
                    plsc.store_scatter(
                        ob, [lanes, jnp.full((EDGE_DIM,), i, jnp.int32)], v)

                start_out(k, b)

                @pl.when(k + NBUF < n_c)
                def _():
                    start_in(k + NBUF, b)

    for b in range(NBUF):
        k_last = n_c - 1 - ((n_c - 1 - b) % NBUF)
        wait_out(k_last, b)


def _edge_update(pr, ps, ridx, sidx):
    mesh = plsc.VectorSubcoreMesh(core_axis_name="c", subcore_axis_name="s")
    f = pl.kernel(
        _edge_body,
        out_type=jax.ShapeDtypeStruct((2, N_TCOLS, 8, 128), jnp.float32),
        mesh=mesh,
        scratch_types=[
            pltpu.VMEM((MAX_WCHUNKS * CHUNK,), jnp.int32),
            pltpu.VMEM((MAX_WCHUNKS * CHUNK,), jnp.int32),
        ] + [pltpu.VMEM((CHUNK, EDGE_DIM), jnp.float32)] * 4
          + [pltpu.VMEM((EDGE_DIM, CHUNK + 1), jnp.float32)] * 2
          + [pltpu.VMEM_SHARED((N_NODES, EDGE_DIM), jnp.float32)] * 2 + [
            pltpu.SemaphoreType.DMA,
            pltpu.SemaphoreType.DMA,
            pltpu.SemaphoreType.DMA,
            pltpu.SemaphoreType.DMA,
            pltpu.SemaphoreType.DMA,
        ],
        compiler_params=pltpu.CompilerParams(use_tc_tiling_on_sc=False,
                                             needs_layout_passes=False),
    )
    return f(pr, ps, ridx, sidx)


def kernel(x, senders, receivers, W_base, b_base, W_edge, b_edge):
    pr, ps = _node_tables(x, W_base, b_base, W_edge, b_edge)
    out4 = _edge_update(pr, ps, receivers, senders)
    return out4.transpose(1, 3, 0, 2).reshape(N_EDGES, EDGE_DIM)


# speedup vs baseline: 28.7011x; 1.1800x over previous
import functools

import jax
import jax.numpy as jnp
from jax import lax
from jax.experimental import pallas as pl
from jax.experimental.pallas import tpu as pltpu
from jax.experimental.pallas import tpu_sc as plsc

N_NODES = 10000
N_EDGES = 320000
D_FEAT = 128
EDGE_DIM = 16

NC = 2
NS = 16
NW = NC * NS
CHUNK = 512
SUB = CHUNK // 128
N_CHUNKS = N_EDGES // CHUNK
N_TCOLS = N_EDGES // 128
MAX_WCHUNKS = N_CHUNKS // NW + 1
NBUF = 2

ROWS_BLK = N_NODES


def _tables_body(x_ref, wb_ref, bb_ref, wr_ref, ws_ref, br_ref, ps_b_ref,
                 pr_ref, ps_ref):
    t = jnp.tanh(
        jnp.dot(x_ref[...], wb_ref[...], preferred_element_type=jnp.float32)
        + bb_ref[...]
    )
    t_r = t.reshape(ROWS_BLK // 8, 8 * D_FEAT)
    pr_ref[...] = (jnp.dot(t_r, wr_ref[...], preferred_element_type=jnp.float32)
                   + br_ref[...])
    ps_ref[...] = (jnp.dot(t_r, ws_ref[...], preferred_element_type=jnp.float32)
                   + ps_b_ref[...])


def _node_tables(x, W_base, b_base, W_edge, b_edge):
    eye8 = jnp.eye(8, dtype=jnp.float32)
    w_r = jnp.kron(eye8, 2.0 * W_edge[:D_FEAT])
    w_s = jnp.kron(eye8, 2.0 * W_edge[D_FEAT:])
    b_r = jnp.tile(2.0 * b_edge, 8).reshape(1, 128)
    b_s = jnp.zeros((1, 128), jnp.float32)
    grid = (N_NODES // ROWS_BLK,)
    return pl.pallas_call(
        _tables_body,
        grid=grid,
        in_specs=[
            pl.BlockSpec((ROWS_BLK, D_FEAT), lambda i: (i, 0)),
            pl.BlockSpec((D_FEAT, D_FEAT), lambda i: (0, 0)),
            pl.BlockSpec((1, D_FEAT), lambda i: (0, 0)),
            pl.BlockSpec((8 * D_FEAT, 128), lambda i: (0, 0)),
            pl.BlockSpec((8 * D_FEAT, 128), lambda i: (0, 0)),
            pl.BlockSpec((1, 128), lambda i: (0, 0)),
            pl.BlockSpec((1, 128), lambda i: (0, 0)),
        ],
        out_specs=[
            pl.BlockSpec((ROWS_BLK // 8, 128), lambda i: (i, 0)),
            pl.BlockSpec((ROWS_BLK // 8, 128), lambda i: (i, 0)),
        ],
        out_shape=[
            jax.ShapeDtypeStruct((N_NODES // 8, 128), jnp.float32),
            jax.ShapeDtypeStruct((N_NODES // 8, 128), jnp.float32),
        ],
    )(x, W_base, b_base.reshape(1, D_FEAT), w_r, w_s, b_r, b_s)


def _edge_body(pr_hbm, ps_hbm, ridx_hbm, sidx_hbm, out_hbm,
               ridx_v, sidx_v,
               rbuf0, rbuf1, sbuf0, sbuf1, obuf0, obuf1,
               pr_s, ps_s,
               sem_i0, sem_i1, sem_o0, sem_o1, sem_t):
    rbufs, sbufs, obufs = [rbuf0, rbuf1], [sbuf0, sbuf1], [obuf0, obuf1]
    sem_is, sem_os = [sem_i0, sem_i1], [sem_o0, sem_o1]
    sid = lax.axis_index("s")
    wid = sid * NC + lax.axis_index("c")

    @pl.when(sid == 0)
    def _():
        pltpu.async_copy(pr_hbm, pr_s, sem_t)
        pltpu.async_copy(ps_hbm, ps_s, sem_t)
    lo_c = lax.shift_right_logical(N_CHUNKS * wid, 5)
    hi_c = lax.shift_right_logical(N_CHUNKS * (wid + 1), 5)
    n_c = hi_c - lo_c
    e_lo = lo_c * CHUNK
    pltpu.sync_copy(ridx_hbm.at[pl.ds(e_lo, MAX_WCHUNKS * CHUNK)], ridx_v)
    pltpu.sync_copy(sidx_hbm.at[pl.ds(e_lo, MAX_WCHUNKS * CHUNK)], sidx_v)
    lanes = jnp.arange(EDGE_DIM, dtype=jnp.int32)

    @pl.when(sid == 0)
    def _():
        pltpu.make_async_copy(pr_hbm, pr_s, sem_t).wait()
        pltpu.make_async_copy(ps_hbm, ps_s, sem_t).wait()

    plsc.subcore_barrier()

    def start_in(k, b):
        idx_r = ridx_v.at[pl.ds(k * CHUNK, CHUNK)]
        idx_s = sidx_v.at[pl.ds(k * CHUNK, CHUNK)]
        pltpu.async_copy(pr_s.at[idx_r], rbufs[b], sem_is[b])
        pltpu.async_copy(ps_s.at[idx_s], sbufs[b], sem_is[b])

    def wait_in(k, b):
        idx_r = ridx_v.at[pl.ds(k * CHUNK, CHUNK)]
        pltpu.make_async_copy(pr_s.at[idx_r], rbufs[b], sem_is[b]).wait()
        pltpu.make_async_copy(pr_s.at[idx_r], sbufs[b], sem_is[b]).wait()

    def start_out(k, b):
        c0 = (lo_c + k) * SUB
        for fr in range(2):
            for sub in range(SUB):
                pltpu.async_copy(
                    obufs[b].at[pl.ds(fr * 8, 8), pl.ds(sub * 128, 128)],
                    out_hbm.at[fr, c0 + sub], sem_os[b])

    def wait_out(k, b):
        c0 = (lo_c + k) * SUB
        for fr in range(2):
            for sub in range(SUB):
                pltpu.make_async_copy(
                    obufs[b].at[pl.ds(fr * 8, 8), pl.ds(sub * 128, 128)],
                    out_hbm.at[fr, c0 + sub], sem_os[b]).wait()

    for b in range(NBUF):
        start_in(b, b)

    @pl.loop(0, MAX_WCHUNKS + (-MAX_WCHUNKS) % NBUF, step=NBUF)
    def outer(k0):
        for b in range(NBUF):
            k = k0 + b

            @pl.when(k < n_c)
            def _():
                wait_in(k, b)

                @pl.when(k >= NBUF)
                def _():
                    wait_out(k - NBUF, b)

                rb, sb, ob = rbufs[b], sbufs[b], obufs[b]

                @plsc.parallel_loop(0, CHUNK, unroll=8)
                def rows(i):
                    e = jnp.exp(rb[i] + sb[i])
                    v = 1.0 - 2.0 / (e + 1.0)
                    plsc.store_scatter(
                        ob, [lanes, jnp.full((EDGE_DIM,), i, jnp.int32)], v)

                start_out(k, b)

                @pl.when(k + NBUF < n_c)
                def _():
                    start_in(k + NBUF, b)

    for b in range(NBUF):
        k_last = n_c - 1 - ((n_c - 1 - b) % NBUF)
        wait_out(k_last, b)


def _edge_update(pr, ps, ridx, sidx):
    mesh = plsc.VectorSubcoreMesh(core_axis_name="c", subcore_axis_name="s")
    f = pl.kernel(
        _edge_body,
        out_type=jax.ShapeDtypeStruct((2, N_TCOLS, 8, 128), jnp.float32),
        mesh=mesh,
        scratch_types=[
            pltpu.VMEM((MAX_WCHUNKS * CHUNK,), jnp.int32),
            pltpu.VMEM((MAX_WCHUNKS * CHUNK,), jnp.int32),
        ] + [pltpu.VMEM((CHUNK, EDGE_DIM), jnp.float32)] * 4
          + [pltpu.VMEM((EDGE_DIM, CHUNK + 1), jnp.float32)] * 2
          + [pltpu.VMEM_SHARED((N_NODES, EDGE_DIM), jnp.float32)] * 2 + [
            pltpu.SemaphoreType.DMA,
            pltpu.SemaphoreType.DMA,
            pltpu.SemaphoreType.DMA,
            pltpu.SemaphoreType.DMA,
            pltpu.SemaphoreType.DMA,
        ],
        compiler_params=pltpu.CompilerParams(use_tc_tiling_on_sc=False,
                                             needs_layout_passes=False),
    )
    return f(pr, ps, ridx, sidx)


def kernel(x, senders, receivers, W_base, b_base, W_edge, b_edge):
    pr_c, ps_c = _node_tables(x, W_base, b_base, W_edge, b_edge)
    pr = pr_c.reshape(N_NODES, EDGE_DIM)
    ps = ps_c.reshape(N_NODES, EDGE_DIM)
    out4 = _edge_update(pr, ps, receivers, senders)
    return out4.transpose(1, 3, 0, 2).reshape(N_EDGES, EDGE_DIM)
